# baseline (speedup 1.0000x reference)
import functools

import jax
import jax.numpy as jnp
from jax import lax
from jax.experimental import pallas as pl
from jax.experimental.pallas import tpu as pltpu
from jax.experimental.pallas import tpu_sc as plsc

_XDIM = 64
_PAD = 128
_CAP = 1000000
_BSZ = 16384
_NC = 2
_NS = 16
_NW = _NC * _NS
_BPW = _BSZ // _NW
_CHUNK = 128
_NCHUNK = _BPW // _CHUNK

_mesh = plsc.VectorSubcoreMesh(core_axis_name="c", subcore_axis_name="s")


@functools.partial(
    pl.kernel,
    mesh=_mesh,
    compiler_params=pltpu.CompilerParams(
        use_tc_tiling_on_sc=True, needs_layout_passes=False
    ),
    out_type=(
        jax.ShapeDtypeStruct((_BSZ, _PAD), jnp.float32),
        jax.ShapeDtypeStruct((_BSZ,), jnp.int32),
    ),
    scratch_types=[
        pltpu.VMEM((_BPW,), jnp.int32),
        pltpu.VMEM((_BPW, _PAD), jnp.float32),
        pltpu.VMEM((_BPW,), jnp.int32),
        pltpu.SemaphoreType.DMA,
        pltpu.SemaphoreType.DMA,
    ],
)
def _fetch(idx_hbm, xp_hbm, my_hbm, out_x, out_y, idx_v, xbuf, y_v,
           sem_x, sem_y):
    wid = lax.axis_index("s") * _NC + lax.axis_index("c")
    base = wid * _BPW
    pltpu.sync_copy(idx_hbm.at[pl.ds(base, _BPW)], idx_v)

    y_copies = []
    for j in range(_NCHUNK):
        y_copies.append(
            pltpu.async_copy(
                my_hbm.at[idx_v.at[pl.ds(j * _CHUNK, _CHUNK)]],
                y_v.at[pl.ds(j * _CHUNK, _CHUNK)],
                sem_y,
            )
        )

    x_copies = []
    for j in range(_NCHUNK):
        x_copies.append(
            pltpu.async_copy(
                xp_hbm.at[idx_v.at[pl.ds(j * _CHUNK, _CHUNK)]],
                xbuf.at[pl.ds(j * _CHUNK, _CHUNK), :],
                sem_x,
            )
        )
    for c in x_copies:
        c.wait()
    for c in y_copies:
        c.wait()
    pltpu.sync_copy(xbuf, out_x.at[pl.ds(base, _BPW), :])
    pltpu.sync_copy(y_v, out_y.at[pl.ds(base, _BPW)])


def kernel(inputs, idx, mems_x, mems_y):
    del inputs
    xp = jnp.concatenate(
        [mems_x, jnp.zeros((_CAP, _PAD - _XDIM), jnp.float32)], axis=1
    )
    out_xp, res_y = _fetch(idx, xp, mems_y)
    return (out_xp[:, :_XDIM], res_y)


# speedup vs baseline: 1.0019x; 1.0019x over previous
import functools

import jax
import jax.numpy as jnp
from jax import lax
from jax.experimental import pallas as pl
from jax.experimental.pallas import tpu as pltpu
from jax.experimental.pallas import tpu_sc as plsc

_XDIM = 64
_PAD = 128
_CAP = 1000000
_BSZ = 16384
_NC = 2
_NS = 16
_NW = _NC * _NS
_BPW = _BSZ // _NW
_CHUNK = 128
_NCHUNK = _BPW // _CHUNK

_mesh = plsc.VectorSubcoreMesh(core_axis_name="c", subcore_axis_name="s")


@functools.partial(
    pl.kernel,
    mesh=_mesh,
    compiler_params=pltpu.CompilerParams(
        use_tc_tiling_on_sc=True, needs_layout_passes=False
    ),
    out_type=(
        jax.ShapeDtypeStruct((_BSZ, _PAD), jnp.float32),
        jax.ShapeDtypeStruct((_BSZ,), jnp.int32),
    ),
    scratch_types=[
        pltpu.VMEM((_BPW,), jnp.int32),
        pltpu.VMEM((_BPW, _PAD), jnp.float32),
        pltpu.VMEM((_BPW,), jnp.int32),
        pltpu.SemaphoreType.DMA,
        pltpu.SemaphoreType.DMA,
    ],
)
def _fetch(idx_hbm, xp_hbm, my_hbm, out_x, out_y, idx_v, xbuf, y_v,
           sem_x, sem_y):
    wid = lax.axis_index("s") * _NC + lax.axis_index("c")
    base = wid * _BPW
    pltpu.sync_copy(idx_hbm.at[pl.ds(base, _BPW)], idx_v)

    y_copies = []
    for j in range(_NCHUNK):
        y_copies.append(
            pltpu.async_copy(
                my_hbm.at[idx_v.at[pl.ds(j * _CHUNK, _CHUNK)]],
                y_v.at[pl.ds(j * _CHUNK, _CHUNK)],
                sem_y,
            )
        )

    x_copies = []
    for j in range(_NCHUNK):
        x_copies.append(
            pltpu.async_copy(
                xp_hbm.at[idx_v.at[pl.ds(j * _CHUNK, _CHUNK)]],
                xbuf.at[pl.ds(j * _CHUNK, _CHUNK), :],
                sem_x,
            )
        )
    for c in x_copies:
        c.wait()
    for c in y_copies:
        c.wait()
    pltpu.sync_copy(xbuf, out_x.at[pl.ds(base, _BPW), :])
    pltpu.sync_copy(y_v, out_y.at[pl.ds(base, _BPW)])


def kernel(inputs, idx, mems_x, mems_y):
    del inputs
    xp = jnp.pad(mems_x, ((0, 0), (0, _PAD - _XDIM)))
    out_xp, res_y = _fetch(idx, xp, mems_y)
    return (out_xp[:, :_XDIM], res_y)
